# baseline (speedup 1.0000x reference)
import functools

import jax
import jax.numpy as jnp
from jax import lax
from jax.experimental import pallas as pl
from jax.experimental.pallas import tpu as pltpu
from jax.experimental.pallas import tpu_sc as plsc

_PNUM = 128
_BATCH = 1024
_NCHUNK = _PNUM // 16
_FLUSH = 8


def _make_sc_kernel(n_workers, b_per_w):
    mesh = plsc.VectorSubcoreMesh(core_axis_name="c", subcore_axis_name="s")

    @functools.partial(
        pl.kernel,
        mesh=mesh,
        out_type=jax.ShapeDtypeStruct((n_workers, 16), jnp.float32),
        scratch_types=[
            pltpu.VMEM((b_per_w * _PNUM,), jnp.int32),
            pltpu.VMEM((b_per_w * 2 * _PNUM,), jnp.int32),
            pltpu.VMEM((16,), jnp.float32),
        ],
        compiler_params=pltpu.CompilerParams(needs_layout_passes=False),
    )
    def sc_kernel(p_hbm, g_hbm, out_hbm, p_v, g_v, out_v):
        nc = 2
        wid = lax.axis_index("s") * nc + lax.axis_index("c")
        base = wid * b_per_w
        pltpu.sync_copy(p_hbm.at[pl.ds(base * _PNUM, b_per_w * _PNUM)], p_v)
        pltpu.sync_copy(
            g_hbm.at[pl.ds(base * 2 * _PNUM, b_per_w * 2 * _PNUM)], g_v)

        lane = jnp.arange(16, dtype=jnp.int32)
        zero16 = jnp.zeros((16,), jnp.int32)
        one_bf = jnp.bfloat16(1.0)
        half_bf = jnp.bfloat16(0.5)

        def batch_body(b, bacc):
            gbase = b * 2 * _PNUM
            pbase = b * _PNUM

            def outer_body(jo, faccs):
                j0 = jo * _FLUSH
                baccs = [jnp.zeros((32,), jnp.bfloat16)
                         for _ in range(_NCHUNK)]
                for jj in range(_FLUSH):
                    j = j0 + jj
                    sidx = zero16 + (pbase + j)
                    pv = plsc.bitcast(plsc.load_gather(p_v, [sidx]),
                                      jnp.bfloat16)
                    idx0 = gbase + j + lane
                    for g in range(_NCHUNK):
                        gv = plsc.bitcast(
                            plsc.load_gather(g_v, [idx0 + g * 16]),
                            jnp.bfloat16)
                        d = pv - gv
                        ad = jnp.abs(d)
                        m = jnp.minimum(ad, one_bf)
                        baccs[g] = baccs[g] + m * (ad - half_bf * m)
                out = []
                for g in range(_NCHUNK):
                    lo, hi = plsc.unpack(
                        baccs[g], format=plsc.PackFormat.INTERLEAVED,
                        preferred_element_type=jnp.float32)
                    out.append(faccs[g] + lo + hi)
                return tuple(out)

            faccs = lax.fori_loop(
                0, _PNUM // _FLUSH, outer_body,
                tuple(jnp.zeros((16,), jnp.float32) for _ in range(_NCHUNK)))
            m = faccs[0]
            for g in range(1, _NCHUNK):
                m = jnp.minimum(m, faccs[g])
            return bacc + jnp.min(m)

        bacc = lax.fori_loop(0, b_per_w, batch_body, jnp.float32(0.0))
        out_v[...] = jnp.zeros((16,), jnp.float32) + bacc
        pltpu.sync_copy(out_v, out_hbm.at[wid])

    return sc_kernel


@jax.jit
def kernel(pred, gt):
    n_workers = 32
    b_per_w = _BATCH // n_workers
    p_words = lax.bitcast_convert_type(
        pred.astype(jnp.bfloat16), jnp.int32).reshape(-1)
    gt2 = jnp.concatenate([gt, gt], axis=1).astype(jnp.bfloat16)
    g_words = lax.bitcast_convert_type(gt2, jnp.int32).reshape(-1)
    partials = _make_sc_kernel(n_workers, b_per_w)(p_words, g_words)
    return jnp.sum(partials[:, 0]) * (1.0 / (_BATCH * _PNUM))
